# baseline (speedup 1.0000x reference)
import functools

import jax
import jax.numpy as jnp
from jax.experimental import pallas as pl
from jax.experimental.pallas import tpu as pltpu

_NUM_Q = 4
_DIM = 64
_K = 8192
_COMMIT = 1.0
_T = 256


def _rvq_kernel(x_ref, e_ref, et_ref, qout_ref, loss_ref, perp_ref,
                counts_ref, lacc_ref, e2_ref, *, n_tokens, n_steps):
    pid = pl.program_id(0)

    @pl.when(pid == 0)
    def _init():
        counts_ref[...] = jnp.zeros_like(counts_ref)
        lacc_ref[...] = jnp.zeros_like(lacc_ref)
        for q in range(_NUM_Q):
            eq = e_ref[q]
            e2_ref[q, :] = jnp.sum(eq * eq, axis=0)

    res = x_ref[...]
    qout = jnp.zeros_like(res)
    iota_k = jax.lax.broadcasted_iota(jnp.int32, (_T, _K), 1)
    iota_q = jax.lax.iota(jnp.int32, _NUM_Q)

    for q in range(_NUM_Q):
        eq = e_ref[q]
        etq = et_ref[q]
        f2 = jnp.sum(res * res, axis=1, keepdims=True)
        fe = jax.lax.dot_general(
            res.astype(jnp.bfloat16), eq.astype(jnp.bfloat16),
            (((1,), (0,)), ((), ())),
            preferred_element_type=jnp.float32)
        dist = (f2 - 2.0 * fe) + e2_ref[q, :][None, :]
        idx = jnp.argmax(-dist, axis=1)
        mask = (idx[:, None] == iota_k).astype(jnp.float32)
        quant = jax.lax.dot_general(
            mask, etq, (((1,), (0,)), ((), ())),
            preferred_element_type=jnp.float32,
            precision=jax.lax.Precision.HIGHEST)
        counts_ref[q, :] = counts_ref[q, :] + jnp.sum(mask, axis=0)
        diff = quant - res
        lsum = jnp.sum(diff * diff)
        lacc_ref[...] = lacc_ref[...] + lsum * (iota_q == q).astype(jnp.float32)
        qs = res + (quant - res)
        qout = qout + qs
        res = res - qs

    qout_ref[...] = qout

    @pl.when(pid == n_steps - 1)
    def _fini():
        scale = _COMMIT / float(n_tokens * _DIM)
        loss_ref[...] = lacc_ref[...] * scale
        avg = counts_ref[...] * (1.0 / float(n_tokens))
        ent = jnp.sum(avg * jnp.log(avg + 1e-10), axis=1)
        perp_ref[...] = jnp.exp(-ent)


@jax.jit
def kernel(x, embeds):
    b, t, d = x.shape
    n_tokens = b * t
    n_steps = n_tokens // _T
    xf = x.reshape(n_tokens, d)
    embeds_t = jnp.swapaxes(embeds, 1, 2)

    grid = (n_steps,)
    qout, losses, perps = pl.pallas_call(
        functools.partial(_rvq_kernel, n_tokens=n_tokens, n_steps=n_steps),
        grid=grid,
        in_specs=[
            pl.BlockSpec((_T, _DIM), lambda i: (i, 0)),
            pl.BlockSpec((_NUM_Q, _DIM, _K), lambda i: (0, 0, 0)),
            pl.BlockSpec((_NUM_Q, _K, _DIM), lambda i: (0, 0, 0)),
        ],
        out_specs=[
            pl.BlockSpec((_T, _DIM), lambda i: (i, 0)),
            pl.BlockSpec((_NUM_Q,), lambda i: (0,)),
            pl.BlockSpec((_NUM_Q,), lambda i: (0,)),
        ],
        out_shape=[
            jax.ShapeDtypeStruct((n_tokens, d), jnp.float32),
            jax.ShapeDtypeStruct((_NUM_Q,), jnp.float32),
            jax.ShapeDtypeStruct((_NUM_Q,), jnp.float32),
        ],
        scratch_shapes=[
            pltpu.VMEM((_NUM_Q, _K), jnp.float32),
            pltpu.VMEM((_NUM_Q,), jnp.float32),
            pltpu.VMEM((_NUM_Q, _K), jnp.float32),
        ],
        compiler_params=pltpu.CompilerParams(
            dimension_semantics=("arbitrary",),
        ),
    )(xf, embeds, embeds_t)
    return qout.reshape(b, t, d), losses, perps


# speedup vs baseline: 3.1451x; 3.1451x over previous
import functools

import jax
import jax.numpy as jnp
from jax.experimental import pallas as pl
from jax.experimental.pallas import tpu as pltpu

_NUM_Q = 4
_DIM = 64
_K = 8192
_COMMIT = 1.0
_T = 256


def _rvq_kernel(x_ref, e_ref, et_ref, qout_ref, loss_ref, perp_ref,
                counts_ref, lacc_ref, e2_ref, ebf_ref,
                ethi_ref, etmid_ref, *, n_tokens, n_steps):
    pid = pl.program_id(0)

    @pl.when(pid == 0)
    def _init():
        counts_ref[...] = jnp.zeros_like(counts_ref)
        lacc_ref[...] = jnp.zeros_like(lacc_ref)
        for q in range(_NUM_Q):
            eq = e_ref[q]
            e2_ref[q, :] = jnp.sum(eq * eq, axis=0)
            ebf_ref[q] = eq.astype(jnp.bfloat16)
            etq = et_ref[q]
            hi = etq.astype(jnp.bfloat16)
            r1 = etq - hi.astype(jnp.float32)
            mid = r1.astype(jnp.bfloat16)
            ethi_ref[q] = hi
            etmid_ref[q] = mid

    res = x_ref[...]
    qout = jnp.zeros_like(res)
    iota_k = jax.lax.broadcasted_iota(jnp.int32, (_T, _K), 1)
    iota_q = jax.lax.iota(jnp.int32, _NUM_Q)

    for q in range(_NUM_Q):
        f2 = jnp.sum(res * res, axis=1, keepdims=True)
        fe = jax.lax.dot_general(
            res.astype(jnp.bfloat16), ebf_ref[q],
            (((1,), (0,)), ((), ())),
            preferred_element_type=jnp.float32)
        dist = (f2 - 2.0 * fe) + e2_ref[q, :][None, :]
        idx = jnp.argmax(-dist, axis=1)
        maskf = (idx[:, None] == iota_k)
        mask = maskf.astype(jnp.bfloat16)
        dots = [jax.lax.dot_general(
            mask, t_ref[q], (((1,), (0,)), ((), ())),
            preferred_element_type=jnp.float32)
            for t_ref in (ethi_ref, etmid_ref)]
        quant = dots[0] + dots[1]
        counts_ref[q, :] = counts_ref[q, :] + jnp.sum(
            maskf.astype(jnp.float32), axis=0)
        diff = quant - res
        lsum = jnp.sum(diff * diff)
        lacc_ref[...] = lacc_ref[...] + lsum * (iota_q == q).astype(jnp.float32)
        qs = res + (quant - res)
        qout = qout + qs
        res = res - qs

    qout_ref[...] = qout

    @pl.when(pid == n_steps - 1)
    def _fini():
        scale = _COMMIT / float(n_tokens * _DIM)
        loss_ref[...] = lacc_ref[...] * scale
        avg = counts_ref[...] * (1.0 / float(n_tokens))
        ent = jnp.sum(avg * jnp.log(avg + 1e-10), axis=1)
        perp_ref[...] = jnp.exp(-ent)


@jax.jit
def kernel(x, embeds):
    b, t, d = x.shape
    n_tokens = b * t
    n_steps = n_tokens // _T
    xf = x.reshape(n_tokens, d)
    embeds_t = jnp.swapaxes(embeds, 1, 2)

    grid = (n_steps,)
    qout, losses, perps = pl.pallas_call(
        functools.partial(_rvq_kernel, n_tokens=n_tokens, n_steps=n_steps),
        grid=grid,
        in_specs=[
            pl.BlockSpec((_T, _DIM), lambda i: (i, 0)),
            pl.BlockSpec((_NUM_Q, _DIM, _K), lambda i: (0, 0, 0)),
            pl.BlockSpec((_NUM_Q, _K, _DIM), lambda i: (0, 0, 0)),
        ],
        out_specs=[
            pl.BlockSpec((_T, _DIM), lambda i: (i, 0)),
            pl.BlockSpec((_NUM_Q,), lambda i: (0,)),
            pl.BlockSpec((_NUM_Q,), lambda i: (0,)),
        ],
        out_shape=[
            jax.ShapeDtypeStruct((n_tokens, d), jnp.float32),
            jax.ShapeDtypeStruct((_NUM_Q,), jnp.float32),
            jax.ShapeDtypeStruct((_NUM_Q,), jnp.float32),
        ],
        scratch_shapes=[
            pltpu.VMEM((_NUM_Q, _K), jnp.float32),
            pltpu.VMEM((_NUM_Q,), jnp.float32),
            pltpu.VMEM((_NUM_Q, _K), jnp.float32),
            pltpu.VMEM((_NUM_Q, _DIM, _K), jnp.bfloat16),
            pltpu.VMEM((_NUM_Q, _K, _DIM), jnp.bfloat16),
            pltpu.VMEM((_NUM_Q, _K, _DIM), jnp.bfloat16),
        ],
        compiler_params=pltpu.CompilerParams(
            dimension_semantics=("arbitrary",),
        ),
    )(xf, embeds, embeds_t)
    return qout.reshape(b, t, d), losses, perps
